# Initial kernel scaffold; baseline (speedup 1.0000x reference)
import jax
import jax.numpy as jnp
from jax.experimental import pallas as pl


def kernel(x, edge_index, W11, b11, W12, b12, W21, b21, W22, b22):
    raise NotImplementedError("write your pallas kernel here")


# speedup vs baseline: 1.2277x; 1.2277x over previous
import jax
import jax.numpy as jnp
from jax import lax
from jax.experimental import pallas as pl
from jax.experimental.pallas import tpu as pltpu
from jax.experimental.pallas import tpu_sc as plsc

N = 10000
E = 320000
D = 128
H = 64

NC = 2
NS = 16
NW = NC * NS
EPW = E // NW
GCH = 80
NGCH = EPW // GCH

NPT_REAL = 313
NPT = 320
DUMMY_SLOT = 316
CAP = 11200
SCH = 80
NSCH = CAP // SCH
DB = 2000

_sc_mesh = plsc.VectorSubcoreMesh(core_axis_name="c", subcore_axis_name="s")


def _wid():
    return lax.axis_index("s") * NC + lax.axis_index("c")


def _compact_body(dst_h, ids_hbm, dloc_hbm, dbuf, ids_v, dloc_v):
    wid = _wid()
    lo = wid * NPT_REAL

    zero16 = jnp.zeros((16,), jnp.int32)
    dum16 = jnp.full((16,), DUMMY_SLOT, jnp.int32)

    def pre(i, c):
        ids_v[pl.ds(i * 16, 16)] = zero16
        dloc_v[pl.ds(i * 16, 16)] = dum16
        return c

    lax.fori_loop(0, CAP // 16, pre, 0)

    lanes = lax.iota(jnp.int32, 16)

    def outer(c, off):
        pltpu.sync_copy(dst_h.at[pl.ds(c * DB, DB)], dbuf)

        def inner(j, off):
            v = dbuf[pl.ds(j * 16, 16)]
            m = (v >= lo) & (v < lo + NPT_REAL) & (off < CAP - 15)
            base = c * DB + j * 16
            cs = plsc.cumsum(m.astype(jnp.int32))
            pos = off + cs - 1
            plsc.store_scatter(ids_v, [pos], lanes + base, mask=m)
            plsc.store_scatter(dloc_v, [pos], v - lo, mask=m)
            return off + cs[15]

        return lax.fori_loop(0, DB // 16, inner, off)

    lax.fori_loop(0, E // DB, outer, 0)
    pltpu.sync_copy(ids_v, ids_hbm.at[wid])
    pltpu.sync_copy(dloc_v, dloc_hbm.at[wid])


_sc_params = pltpu.CompilerParams(
    needs_layout_passes=False, use_tc_tiling_on_sc=False)

_compact = pl.kernel(
    _compact_body,
    out_type=(
        jax.ShapeDtypeStruct((NW, CAP), jnp.int32),
        jax.ShapeDtypeStruct((NW, CAP), jnp.int32),
    ),
    mesh=_sc_mesh,
    compiler_params=_sc_params,
    scratch_types=[
        pltpu.VMEM((DB,), jnp.int32),
        pltpu.VMEM((CAP,), jnp.int32),
        pltpu.VMEM((CAP,), jnp.int32),
    ],
)


def _gather_body(dst_h, src_h, a_hbm, b_hbm, ga_hbm, gb_hbm, idxd, idxs, bufa, bufb, sem):
    wid = _wid()
    base = wid * EPW

    def chunk(i, c):
        eb = base + i * GCH
        pltpu.sync_copy(dst_h.at[pl.ds(eb, GCH)], idxd)
        pltpu.sync_copy(src_h.at[pl.ds(eb, GCH)], idxs)
        ca = pltpu.async_copy(a_hbm.at[idxd], bufa, sem)
        cb = pltpu.async_copy(b_hbm.at[idxs], bufb, sem)
        ca.wait()
        cb.wait()
        pltpu.sync_copy(bufa, ga_hbm.at[pl.ds(eb, GCH)])
        pltpu.sync_copy(bufb, gb_hbm.at[pl.ds(eb, GCH)])
        return c

    lax.fori_loop(0, NGCH, chunk, 0)


_gather = pl.kernel(
    _gather_body,
    out_type=(
        jax.ShapeDtypeStruct((E, H), jnp.float32),
        jax.ShapeDtypeStruct((E, H), jnp.float32),
    ),
    mesh=_sc_mesh,
    compiler_params=_sc_params,
    scratch_types=[
        pltpu.VMEM((GCH,), jnp.int32),
        pltpu.VMEM((GCH,), jnp.int32),
        pltpu.VMEM((GCH, H), jnp.float32),
        pltpu.VMEM((GCH, H), jnp.float32),
        pltpu.SemaphoreType.DMA,
    ],
)


def _make_scatter(layer2: bool):
    def body(ids_hbm, dloc_hbm, y_hbm, out_hbm, idxb, dlb, ybuf, acc, sem):
        wid = _wid()
        lo = wid * NPT_REAL
        initv = jnp.full((16,), -jnp.inf if layer2 else 0.0, jnp.float32)

        def ini(i, c):
            for cc in range(4):
                acc[i, pl.ds(cc * 16, 16)] = initv
            return c

        lax.fori_loop(0, NPT, ini, 0)

        def chunk(k, c):
            pltpu.sync_copy(ids_hbm.at[wid, pl.ds(k * SCH, SCH)], idxb)
            pltpu.sync_copy(dloc_hbm.at[wid, pl.ds(k * SCH, SCH)], dlb)
            pltpu.async_copy(y_hbm.at[idxb], ybuf, sem).wait()

            def edge16(g, c2):
                dvec = dlb[pl.ds(g * 16, 16)]
                for u in range(16):
                    r = g * 16 + u
                    d = dvec[u]
                    for cc in range(4):
                        sl = pl.ds(cc * 16, 16)
                        acc[d, sl] = jnp.maximum(acc[d, sl], ybuf[r, sl])
                return c2

            lax.fori_loop(0, SCH // 16, edge16, 0)
            return c

        lax.fori_loop(0, NSCH, chunk, 0)

        if layer2:
            ninf = jnp.float32(-jnp.inf)

            def fix(i, c):
                for cc in range(4):
                    sl = pl.ds(cc * 16, 16)
                    v = acc[i, sl]
                    acc[i, sl] = jnp.where(v == ninf, jnp.float32(0.0), v)
                return c

            lax.fori_loop(0, NPT_REAL, fix, 0)

        last = N - (NW - 1) * NPT_REAL

        @pl.when(wid == NW - 1)
        def _():
            pltpu.sync_copy(acc.at[pl.ds(0, last)], out_hbm.at[pl.ds(lo, last)])

        @pl.when(wid != NW - 1)
        def _():
            pltpu.sync_copy(acc.at[pl.ds(0, NPT_REAL)],
                            out_hbm.at[pl.ds(lo, NPT_REAL)])

    return pl.kernel(
        body,
        out_type=jax.ShapeDtypeStruct((N, H), jnp.float32),
        mesh=_sc_mesh,
        compiler_params=_sc_params,
        scratch_types=[
            pltpu.VMEM((SCH,), jnp.int32),
            pltpu.VMEM((SCH,), jnp.int32),
            pltpu.VMEM((SCH, H), jnp.float32),
            pltpu.VMEM((NPT, H), jnp.float32),
            pltpu.SemaphoreType.DMA,
        ],
    )


_scatter1 = _make_scatter(layer2=False)
_scatter2 = _make_scatter(layer2=True)


def _precompute(x, W, Din):
    def body(x_ref, w_ref, a_ref, b_ref):
        xv = x_ref[...]
        wa = w_ref[0:Din, :]
        wb = w_ref[Din:2 * Din, :]
        a_ref[...] = jnp.dot(xv, wa - wb, preferred_element_type=jnp.float32,
                            precision=lax.Precision.HIGHEST)
        b_ref[...] = jnp.dot(xv, wb, preferred_element_type=jnp.float32,
                            precision=lax.Precision.HIGHEST)

    return pl.pallas_call(
        body,
        out_shape=(
            jax.ShapeDtypeStruct((N, H), jnp.float32),
            jax.ShapeDtypeStruct((N, H), jnp.float32),
        ),
    )(x, W)


MB = 2000


def _mlp(ga, gb, b1, W2, b2):
    def body(ga_ref, gb_ref, b1_ref, w2_ref, b2_ref, y_ref):
        z = ga_ref[...] + gb_ref[...] + b1_ref[...]
        h = jnp.maximum(z, 0.0)
        y_ref[...] = (jnp.dot(h, w2_ref[...], preferred_element_type=jnp.float32,
                             precision=lax.Precision.HIGHEST)
                      + b2_ref[...])

    return pl.pallas_call(
        body,
        grid=(E // MB,),
        in_specs=[
            pl.BlockSpec((MB, H), lambda i: (i, 0)),
            pl.BlockSpec((MB, H), lambda i: (i, 0)),
            pl.BlockSpec((1, H), lambda i: (0, 0)),
            pl.BlockSpec((H, H), lambda i: (0, 0)),
            pl.BlockSpec((1, H), lambda i: (0, 0)),
        ],
        out_specs=pl.BlockSpec((MB, H), lambda i: (i, 0)),
        out_shape=jax.ShapeDtypeStruct((E, H), jnp.float32),
    )(ga, gb, b1, W2, b2)


def kernel(x, edge_index, W11, b11, W12, b12, W21, b21, W22, b22):
    src = edge_index[0]
    dst = edge_index[1]
    ids, dloc = _compact(dst)

    A1, B1 = _precompute(x, W11, D)
    GA1, GB1 = _gather(dst, src, A1, B1)
    Y1 = _mlp(GA1, GB1, b11.reshape(1, H), W12, b12.reshape(1, H))
    h = _scatter1(ids, dloc, Y1)

    A2, B2 = _precompute(h, W21, H)
    GA2, GB2 = _gather(dst, src, A2, B2)
    Y2 = _mlp(GA2, GB2, b21.reshape(1, H), W22, b22.reshape(1, H))
    out = _scatter2(ids, dloc, Y2)
    return out


# speedup vs baseline: 1.6738x; 1.3633x over previous
import jax
import jax.numpy as jnp
from jax import lax
from jax.experimental import pallas as pl
from jax.experimental.pallas import tpu as pltpu
from jax.experimental.pallas import tpu_sc as plsc

N = 10000
E = 320000
D = 128
H = 64

NC = 2
NS = 16
NW = NC * NS
EPW = E // NW
GCH = 80
NGCH = EPW // GCH

NPT_REAL = 313
NPT = 320
DUMMY_SLOT = 316
CAP = 11200
SCH = 80
NSCH = CAP // SCH
DB = 2000

_sc_mesh = plsc.VectorSubcoreMesh(core_axis_name="c", subcore_axis_name="s")


def _wid():
    return lax.axis_index("s") * NC + lax.axis_index("c")


def _compact_body(dst_h, ids_hbm, dloc_hbm, dbuf, ids_v, dloc_v):
    wid = _wid()
    lo = wid * NPT_REAL

    zero16 = jnp.zeros((16,), jnp.int32)
    dum16 = jnp.full((16,), DUMMY_SLOT, jnp.int32)

    def pre(i, c):
        ids_v[pl.ds(i * 16, 16)] = zero16
        dloc_v[pl.ds(i * 16, 16)] = dum16
        return c

    lax.fori_loop(0, CAP // 16, pre, 0)

    lanes = lax.iota(jnp.int32, 16)

    def outer(c, off):
        pltpu.sync_copy(dst_h.at[pl.ds(c * DB, DB)], dbuf)

        def inner(j, off):
            v = dbuf[pl.ds(j * 16, 16)]
            m = (v >= lo) & (v < lo + NPT_REAL) & (off < CAP - 15)
            base = c * DB + j * 16
            cs = plsc.cumsum(m.astype(jnp.int32))
            pos = off + cs - 1
            plsc.store_scatter(ids_v, [pos], lanes + base, mask=m)
            plsc.store_scatter(dloc_v, [pos], v - lo, mask=m)
            return off + cs[15]

        return lax.fori_loop(0, DB // 16, inner, off)

    lax.fori_loop(0, E // DB, outer, 0)
    pltpu.sync_copy(ids_v, ids_hbm.at[wid])
    pltpu.sync_copy(dloc_v, dloc_hbm.at[wid])


_sc_params = pltpu.CompilerParams(
    needs_layout_passes=False, use_tc_tiling_on_sc=False)

_compact = pl.kernel(
    _compact_body,
    out_type=(
        jax.ShapeDtypeStruct((NW, CAP), jnp.int32),
        jax.ShapeDtypeStruct((NW, CAP), jnp.int32),
    ),
    mesh=_sc_mesh,
    compiler_params=_sc_params,
    scratch_types=[
        pltpu.VMEM((DB,), jnp.int32),
        pltpu.VMEM((CAP,), jnp.int32),
        pltpu.VMEM((CAP,), jnp.int32),
    ],
)


ZCH = 200
SUB = 40
NSUB = ZCH // SUB
NZ = EPW // ZCH


def _gather_body(dst_h, src_h, a_hbm, b_hbm, z_hbm, idxd, idxs, bufa, bufb, sem):
    wid = _wid()
    base = wid * EPW
    pltpu.sync_copy(dst_h.at[pl.ds(base, EPW)], idxd)
    pltpu.sync_copy(src_h.at[pl.ds(base, EPW)], idxs)

    def fire(p, slot):
        off = p * ZCH
        for q in range(NSUB):
            isl = pl.ds(off + q * SUB, SUB)
            bsl = pl.ds(q * SUB, SUB)
            pltpu.async_copy(a_hbm.at[idxd.at[isl]], bufa.at[slot, bsl],
                             sem.at[slot])
            pltpu.async_copy(b_hbm.at[idxs.at[isl]], bufb.at[slot, bsl],
                             sem.at[slot])

    def drain(p, slot):
        off = p * ZCH
        for q in range(NSUB):
            isl = pl.ds(off + q * SUB, SUB)
            bsl = pl.ds(q * SUB, SUB)
            pltpu.make_async_copy(a_hbm.at[idxd.at[isl]], bufa.at[slot, bsl],
                                  sem.at[slot]).wait()
            pltpu.make_async_copy(b_hbm.at[idxs.at[isl]], bufb.at[slot, bsl],
                                  sem.at[slot]).wait()

    fire(0, 0)

    def pair(P, c):
        for b2 in (0, 1):
            p = 2 * P + b2

            @pl.when(p + 1 < NZ)
            def _():
                fire(p + 1, 1 - b2)

            drain(p, b2)

            def addrow(r, c2):
                for cc in range(4):
                    sl = pl.ds(cc * 16, 16)
                    bufa[b2, r, sl] = bufa[b2, r, sl] + bufb[b2, r, sl]
                return c2

            lax.fori_loop(0, ZCH, addrow, 0)
            pltpu.sync_copy(bufa.at[b2], z_hbm.at[pl.ds(base + p * ZCH, ZCH)])
        return c

    lax.fori_loop(0, NZ // 2, pair, 0)


_gather = pl.kernel(
    _gather_body,
    out_type=jax.ShapeDtypeStruct((E, H), jnp.float32),
    mesh=_sc_mesh,
    compiler_params=_sc_params,
    scratch_types=[
        pltpu.VMEM((EPW,), jnp.int32),
        pltpu.VMEM((EPW,), jnp.int32),
        pltpu.VMEM((2, ZCH, H), jnp.float32),
        pltpu.VMEM((2, ZCH, H), jnp.float32),
        pltpu.SemaphoreType.DMA((2,)),
    ],
)


SCH2 = 400
SSUB = 80
NSSUB = SCH2 // SSUB
NSC = CAP // SCH2


def _make_scatter(layer2: bool):
    def body(ids_hbm, dloc_hbm, y_hbm, out_hbm, idxb, dlb, ybuf, acc, sem):
        wid = _wid()
        lo = wid * NPT_REAL
        initv = jnp.full((16,), -jnp.inf if layer2 else 0.0, jnp.float32)

        def ini(i, c):
            for cc in range(4):
                acc[i, pl.ds(cc * 16, 16)] = initv
            return c

        lax.fori_loop(0, NPT, ini, 0)

        pltpu.sync_copy(ids_hbm.at[wid], idxb)
        pltpu.sync_copy(dloc_hbm.at[wid], dlb)

        def fire(p, slot):
            off = p * SCH2
            for q in range(NSSUB):
                isl = pl.ds(off + q * SSUB, SSUB)
                bsl = pl.ds(q * SSUB, SSUB)
                pltpu.async_copy(y_hbm.at[idxb.at[isl]], ybuf.at[slot, bsl],
                                 sem.at[slot])

        def drain(p, slot):
            off = p * SCH2
            for q in range(NSSUB):
                isl = pl.ds(off + q * SSUB, SSUB)
                bsl = pl.ds(q * SSUB, SSUB)
                pltpu.make_async_copy(y_hbm.at[idxb.at[isl]],
                                      ybuf.at[slot, bsl], sem.at[slot]).wait()

        fire(0, 0)

        def pair(P, c):
            for b2 in (0, 1):
                p = 2 * P + b2

                @pl.when(p + 1 < NSC)
                def _():
                    fire(p + 1, 1 - b2)

                drain(p, b2)

                def edge16(g, c2):
                    dvec = dlb[pl.ds(p * SCH2 + g * 16, 16)]
                    for u in range(16):
                        r = g * 16 + u
                        d = dvec[u]
                        for cc in range(4):
                            sl = pl.ds(cc * 16, 16)
                            acc[d, sl] = jnp.maximum(acc[d, sl],
                                                     ybuf[b2, r, sl])
                    return c2

                lax.fori_loop(0, SCH2 // 16, edge16, 0)
            return c

        lax.fori_loop(0, NSC // 2, pair, 0)

        if layer2:
            ninf = jnp.float32(-jnp.inf)

            def fix(i, c):
                for cc in range(4):
                    sl = pl.ds(cc * 16, 16)
                    v = acc[i, sl]
                    acc[i, sl] = jnp.where(v == ninf, jnp.float32(0.0), v)
                return c

            lax.fori_loop(0, NPT_REAL, fix, 0)

        last = N - (NW - 1) * NPT_REAL

        @pl.when(wid == NW - 1)
        def _():
            pltpu.sync_copy(acc.at[pl.ds(0, last)], out_hbm.at[pl.ds(lo, last)])

        @pl.when(wid != NW - 1)
        def _():
            pltpu.sync_copy(acc.at[pl.ds(0, NPT_REAL)],
                            out_hbm.at[pl.ds(lo, NPT_REAL)])

    return pl.kernel(
        body,
        out_type=jax.ShapeDtypeStruct((N, H), jnp.float32),
        mesh=_sc_mesh,
        compiler_params=_sc_params,
        scratch_types=[
            pltpu.VMEM((CAP,), jnp.int32),
            pltpu.VMEM((CAP,), jnp.int32),
            pltpu.VMEM((2, SCH2, H), jnp.float32),
            pltpu.VMEM((NPT, H), jnp.float32),
            pltpu.SemaphoreType.DMA((2,)),
        ],
    )


_scatter1 = _make_scatter(layer2=False)
_scatter2 = _make_scatter(layer2=True)


def _precompute(x, W, Din):
    def body(x_ref, w_ref, a_ref, b_ref):
        xv = x_ref[...]
        wa = w_ref[0:Din, :]
        wb = w_ref[Din:2 * Din, :]
        a_ref[...] = jnp.dot(xv, wa - wb, preferred_element_type=jnp.float32,
                            precision=lax.Precision.HIGHEST)
        b_ref[...] = jnp.dot(xv, wb, preferred_element_type=jnp.float32,
                            precision=lax.Precision.HIGHEST)

    return pl.pallas_call(
        body,
        out_shape=(
            jax.ShapeDtypeStruct((N, H), jnp.float32),
            jax.ShapeDtypeStruct((N, H), jnp.float32),
        ),
    )(x, W)


MB = 2000


def _mlp(z, b1, W2, b2):
    def body(z_ref, b1_ref, w2_ref, b2_ref, y_ref):
        h = jnp.maximum(z_ref[...] + b1_ref[...], 0.0)
        y_ref[...] = (jnp.dot(h, w2_ref[...], preferred_element_type=jnp.float32,
                             precision=lax.Precision.HIGHEST)
                      + b2_ref[...])

    return pl.pallas_call(
        body,
        grid=(E // MB,),
        in_specs=[
            pl.BlockSpec((MB, H), lambda i: (i, 0)),
            pl.BlockSpec((1, H), lambda i: (0, 0)),
            pl.BlockSpec((H, H), lambda i: (0, 0)),
            pl.BlockSpec((1, H), lambda i: (0, 0)),
        ],
        out_specs=pl.BlockSpec((MB, H), lambda i: (i, 0)),
        out_shape=jax.ShapeDtypeStruct((E, H), jnp.float32),
    )(z, b1, W2, b2)


def kernel(x, edge_index, W11, b11, W12, b12, W21, b21, W22, b22):
    src = edge_index[0]
    dst = edge_index[1]
    ids, dloc = _compact(dst)

    A1, B1 = _precompute(x, W11, D)
    Z1 = _gather(dst, src, A1, B1)
    Y1 = _mlp(Z1, b11.reshape(1, H), W12, b12.reshape(1, H))
    h = _scatter1(ids, dloc, Y1)

    A2, B2 = _precompute(h, W21, H)
    Z2 = _gather(dst, src, A2, B2)
    Y2 = _mlp(Z2, b21.reshape(1, H), W22, b22.reshape(1, H))
    out = _scatter2(ids, dloc, Y2)
    return out


# speedup vs baseline: 1.6751x; 1.0008x over previous
import jax
import jax.numpy as jnp
from jax import lax
from jax.experimental import pallas as pl
from jax.experimental.pallas import tpu as pltpu
from jax.experimental.pallas import tpu_sc as plsc

N = 10000
E = 320000
D = 128
H = 64

NC = 2
NS = 16
NW = NC * NS
EPW = E // NW
GCH = 80
NGCH = EPW // GCH

NPT_REAL = 313
NPT = 320
DUMMY_SLOT = 316
CAP = 11200
SCH = 80
NSCH = CAP // SCH
DB = 2000

_sc_mesh = plsc.VectorSubcoreMesh(core_axis_name="c", subcore_axis_name="s")


def _wid():
    return lax.axis_index("s") * NC + lax.axis_index("c")


def _compact_body(dst_h, ids_hbm, dloc_hbm, dbuf, ids_v, dloc_v):
    wid = _wid()
    lo = wid * NPT_REAL

    zero16 = jnp.zeros((16,), jnp.int32)
    dum16 = jnp.full((16,), DUMMY_SLOT, jnp.int32)

    def pre(i, c):
        ids_v[pl.ds(i * 16, 16)] = zero16
        dloc_v[pl.ds(i * 16, 16)] = dum16
        return c

    lax.fori_loop(0, CAP // 16, pre, 0)

    lanes = lax.iota(jnp.int32, 16)

    def outer(c, off):
        pltpu.sync_copy(dst_h.at[pl.ds(c * DB, DB)], dbuf)

        def inner(j, off):
            v = dbuf[pl.ds(j * 16, 16)]
            m = (v >= lo) & (v < lo + NPT_REAL) & (off < CAP - 15)
            base = c * DB + j * 16
            cs = plsc.cumsum(m.astype(jnp.int32))
            pos = off + cs - 1
            plsc.store_scatter(ids_v, [pos], lanes + base, mask=m)
            plsc.store_scatter(dloc_v, [pos], v - lo, mask=m)
            return off + cs[15]

        return lax.fori_loop(0, DB // 16, inner, off)

    lax.fori_loop(0, E // DB, outer, 0)
    pltpu.sync_copy(ids_v, ids_hbm.at[wid])
    pltpu.sync_copy(dloc_v, dloc_hbm.at[wid])


_sc_params = pltpu.CompilerParams(
    needs_layout_passes=False, use_tc_tiling_on_sc=False)

_compact = pl.kernel(
    _compact_body,
    out_type=(
        jax.ShapeDtypeStruct((NW, CAP), jnp.int32),
        jax.ShapeDtypeStruct((NW, CAP), jnp.int32),
    ),
    mesh=_sc_mesh,
    compiler_params=_sc_params,
    scratch_types=[
        pltpu.VMEM((DB,), jnp.int32),
        pltpu.VMEM((CAP,), jnp.int32),
        pltpu.VMEM((CAP,), jnp.int32),
    ],
)


ZCH = 200
SUB = 40
NSUB = ZCH // SUB
NZ = EPW // ZCH


def _gather_body(dst_h, src_h, a_hbm, b_hbm, z_hbm, idxd, idxs, bufa, bufb,
                 sem, semw):
    wid = _wid()
    base = wid * EPW
    pltpu.sync_copy(dst_h.at[pl.ds(base, EPW)], idxd)
    pltpu.sync_copy(src_h.at[pl.ds(base, EPW)], idxs)

    def fire(p, slot):
        off = p * ZCH
        for q in range(NSUB):
            isl = pl.ds(off + q * SUB, SUB)
            bsl = pl.ds(q * SUB, SUB)
            pltpu.async_copy(a_hbm.at[idxd.at[isl]], bufa.at[slot, bsl],
                             sem.at[slot])
            pltpu.async_copy(b_hbm.at[idxs.at[isl]], bufb.at[slot, bsl],
                             sem.at[slot])

    def drain(p, slot):
        off = p * ZCH
        for q in range(NSUB):
            isl = pl.ds(off + q * SUB, SUB)
            bsl = pl.ds(q * SUB, SUB)
            pltpu.make_async_copy(a_hbm.at[idxd.at[isl]], bufa.at[slot, bsl],
                                  sem.at[slot]).wait()
            pltpu.make_async_copy(b_hbm.at[idxs.at[isl]], bufb.at[slot, bsl],
                                  sem.at[slot]).wait()

    def zdesc(p, slot):
        return pltpu.make_async_copy(
            bufa.at[slot], z_hbm.at[pl.ds(base + p * ZCH, ZCH)], semw.at[slot])

    fire(0, 0)

    def pair(P, c):
        for b2 in (0, 1):
            p = 2 * P + b2

            @pl.when(p + 1 < NZ)
            def _():
                @pl.when(p >= 1)
                def _():
                    zdesc(p - 1, 1 - b2).wait()

                fire(p + 1, 1 - b2)

            drain(p, b2)

            def addrow(g, c2):
                for rr in range(4):
                    r = g * 4 + rr
                    for cc in range(4):
                        sl = pl.ds(cc * 16, 16)
                        bufa[b2, r, sl] = bufa[b2, r, sl] + bufb[b2, r, sl]
                return c2

            lax.fori_loop(0, ZCH // 4, addrow, 0)
            zdesc(p, b2).start()
        return c

    lax.fori_loop(0, NZ // 2, pair, 0)
    zdesc(NZ - 2, 0).wait()
    zdesc(NZ - 1, 1).wait()


_gather = pl.kernel(
    _gather_body,
    out_type=jax.ShapeDtypeStruct((E, H), jnp.float32),
    mesh=_sc_mesh,
    compiler_params=_sc_params,
    scratch_types=[
        pltpu.VMEM((EPW,), jnp.int32),
        pltpu.VMEM((EPW,), jnp.int32),
        pltpu.VMEM((2, ZCH, H), jnp.float32),
        pltpu.VMEM((2, ZCH, H), jnp.float32),
        pltpu.SemaphoreType.DMA((2,)),
        pltpu.SemaphoreType.DMA((2,)),
    ],
)


SCH2 = 400
SSUB = 80
NSSUB = SCH2 // SSUB
NSC = CAP // SCH2


def _make_scatter(layer2: bool):
    def body(ids_hbm, dloc_hbm, y_hbm, out_hbm, idxb, dlb, ybuf, acc, sem):
        wid = _wid()
        lo = wid * NPT_REAL
        initv = jnp.full((16,), -jnp.inf if layer2 else 0.0, jnp.float32)

        def ini(i, c):
            for cc in range(4):
                acc[i, pl.ds(cc * 16, 16)] = initv
            return c

        lax.fori_loop(0, NPT, ini, 0)

        pltpu.sync_copy(ids_hbm.at[wid], idxb)
        pltpu.sync_copy(dloc_hbm.at[wid], dlb)

        def fire(p, slot):
            off = p * SCH2
            for q in range(NSSUB):
                isl = pl.ds(off + q * SSUB, SSUB)
                bsl = pl.ds(q * SSUB, SSUB)
                pltpu.async_copy(y_hbm.at[idxb.at[isl]], ybuf.at[slot, bsl],
                                 sem.at[slot])

        def drain(p, slot):
            off = p * SCH2
            for q in range(NSSUB):
                isl = pl.ds(off + q * SSUB, SSUB)
                bsl = pl.ds(q * SSUB, SSUB)
                pltpu.make_async_copy(y_hbm.at[idxb.at[isl]],
                                      ybuf.at[slot, bsl], sem.at[slot]).wait()

        fire(0, 0)

        def pair(P, c):
            for b2 in (0, 1):
                p = 2 * P + b2

                @pl.when(p + 1 < NSC)
                def _():
                    fire(p + 1, 1 - b2)

                drain(p, b2)

                def edge16(g, c2):
                    dvec = dlb[pl.ds(p * SCH2 + g * 16, 16)]
                    for u in range(16):
                        r = g * 16 + u
                        d = dvec[u]
                        for cc in range(4):
                            sl = pl.ds(cc * 16, 16)
                            acc[d, sl] = jnp.maximum(acc[d, sl],
                                                     ybuf[b2, r, sl])
                    return c2

                lax.fori_loop(0, SCH2 // 16, edge16, 0)
            return c

        lax.fori_loop(0, NSC // 2, pair, 0)

        if layer2:
            ninf = jnp.float32(-jnp.inf)

            def fix(i, c):
                for cc in range(4):
                    sl = pl.ds(cc * 16, 16)
                    v = acc[i, sl]
                    acc[i, sl] = jnp.where(v == ninf, jnp.float32(0.0), v)
                return c

            lax.fori_loop(0, NPT_REAL, fix, 0)

        last = N - (NW - 1) * NPT_REAL

        @pl.when(wid == NW - 1)
        def _():
            pltpu.sync_copy(acc.at[pl.ds(0, last)], out_hbm.at[pl.ds(lo, last)])

        @pl.when(wid != NW - 1)
        def _():
            pltpu.sync_copy(acc.at[pl.ds(0, NPT_REAL)],
                            out_hbm.at[pl.ds(lo, NPT_REAL)])

    return pl.kernel(
        body,
        out_type=jax.ShapeDtypeStruct((N, H), jnp.float32),
        mesh=_sc_mesh,
        compiler_params=_sc_params,
        scratch_types=[
            pltpu.VMEM((CAP,), jnp.int32),
            pltpu.VMEM((CAP,), jnp.int32),
            pltpu.VMEM((2, SCH2, H), jnp.float32),
            pltpu.VMEM((NPT, H), jnp.float32),
            pltpu.SemaphoreType.DMA((2,)),
        ],
    )


_scatter1 = _make_scatter(layer2=False)
_scatter2 = _make_scatter(layer2=True)


def _precompute(x, W, Din):
    def body(x_ref, w_ref, a_ref, b_ref):
        xv = x_ref[...]
        wa = w_ref[0:Din, :]
        wb = w_ref[Din:2 * Din, :]
        a_ref[...] = jnp.dot(xv, wa - wb, preferred_element_type=jnp.float32,
                            precision=lax.Precision.HIGHEST)
        b_ref[...] = jnp.dot(xv, wb, preferred_element_type=jnp.float32,
                            precision=lax.Precision.HIGHEST)

    return pl.pallas_call(
        body,
        out_shape=(
            jax.ShapeDtypeStruct((N, H), jnp.float32),
            jax.ShapeDtypeStruct((N, H), jnp.float32),
        ),
    )(x, W)


MB = 2000


def _mlp(z, b1, W2, b2):
    def body(z_ref, b1_ref, w2_ref, b2_ref, y_ref):
        h = jnp.maximum(z_ref[...] + b1_ref[...], 0.0)
        y_ref[...] = (jnp.dot(h, w2_ref[...], preferred_element_type=jnp.float32,
                             precision=lax.Precision.HIGHEST)
                      + b2_ref[...])

    return pl.pallas_call(
        body,
        grid=(E // MB,),
        in_specs=[
            pl.BlockSpec((MB, H), lambda i: (i, 0)),
            pl.BlockSpec((1, H), lambda i: (0, 0)),
            pl.BlockSpec((H, H), lambda i: (0, 0)),
            pl.BlockSpec((1, H), lambda i: (0, 0)),
        ],
        out_specs=pl.BlockSpec((MB, H), lambda i: (i, 0)),
        out_shape=jax.ShapeDtypeStruct((E, H), jnp.float32),
    )(z, b1, W2, b2)


def kernel(x, edge_index, W11, b11, W12, b12, W21, b21, W22, b22):
    src = edge_index[0]
    dst = edge_index[1]
    ids, dloc = _compact(dst)

    A1, B1 = _precompute(x, W11, D)
    Z1 = _gather(dst, src, A1, B1)
    Y1 = _mlp(Z1, b11.reshape(1, H), W12, b12.reshape(1, H))
    h = _scatter1(ids, dloc, Y1)

    A2, B2 = _precompute(h, W21, H)
    Z2 = _gather(dst, src, A2, B2)
    Y2 = _mlp(Z2, b21.reshape(1, H), W22, b22.reshape(1, H))
    out = _scatter2(ids, dloc, Y2)
    return out


# speedup vs baseline: 1.7990x; 1.0740x over previous
import jax
import jax.numpy as jnp
from jax import lax
from jax.experimental import pallas as pl
from jax.experimental.pallas import tpu as pltpu
from jax.experimental.pallas import tpu_sc as plsc

N = 10000
E = 320000
D = 128
H = 64

NC = 2
NS = 16
NW = NC * NS
EPW = E // NW
GCH = 80
NGCH = EPW // GCH

NPT_REAL = 313
NPT = 320
DUMMY_SLOT = 313
CAP = 11200
SCH = 80
NSCH = CAP // SCH
DB = 2000

_sc_mesh = plsc.VectorSubcoreMesh(core_axis_name="c", subcore_axis_name="s")
_sc_params = pltpu.CompilerParams(
    needs_layout_passes=False, use_tc_tiling_on_sc=False)


def _wid():
    return lax.axis_index("s") * NC + lax.axis_index("c")


OFFN = 320
SCAN_BASE = 1


def _compact_body(dst_h, sids_hbm, offs_hbm, dbuf, ids_v, dloc_v, sids_v,
                  cnt_v, offs_v, cur_v):
    wid = _wid()
    lo = wid * NPT_REAL

    zero16 = jnp.zeros((16,), jnp.int32)
    dum16 = jnp.full((16,), DUMMY_SLOT, jnp.int32)

    def pre(i, c):
        ids_v[pl.ds(i * 16, 16)] = zero16
        dloc_v[pl.ds(i * 16, 16)] = dum16
        sids_v[pl.ds(i * 16, 16)] = zero16
        return c

    lax.fori_loop(0, CAP // 16, pre, 0)

    lanes = lax.iota(jnp.int32, 16)

    def outer(c, off):
        pltpu.sync_copy(dst_h.at[pl.ds(c * DB, DB)], dbuf)

        def inner(j, off):
            v = dbuf[pl.ds(j * 16, 16)]
            m = (v >= lo) & (v < lo + NPT_REAL) & (off < CAP - 15)
            base = c * DB + j * 16
            cs = plsc.cumsum(m.astype(jnp.int32))
            pos = off + cs - 1
            plsc.store_scatter(ids_v, [pos], lanes + base, mask=m)
            plsc.store_scatter(dloc_v, [pos], v - lo, mask=m)
            return off + cs[15]

        return lax.fori_loop(0, DB // 16, inner, off)

    lax.fori_loop(0, E // DB, outer, 0)

    def czero(g, c):
        cnt_v[pl.ds(g * 16, 16)] = zero16
        return c

    lax.fori_loop(0, OFFN // 16, czero, 0)

    def count(g, c):
        dv = dloc_v[pl.ds(g * 16, 16)]
        rank, lastm = plsc.scan_count(dv)
        cur = plsc.load_gather(cnt_v, [dv])
        plsc.store_scatter(cnt_v, [dv], cur + rank + (1 - SCAN_BASE),
                           mask=lastm)
        return c

    lax.fori_loop(0, CAP // 16, count, 0)

    def prefix(g, carry):
        cv = cnt_v[pl.ds(g * 16, 16)]
        incl = plsc.cumsum(cv) + carry
        offs_v[pl.ds(g * 16, 16)] = incl - cv
        return incl[15]

    lax.fori_loop(0, OFFN // 16, prefix, 0)

    def ccopy(g, c):
        cur_v[pl.ds(g * 16, 16)] = offs_v[pl.ds(g * 16, 16)]
        return c

    lax.fori_loop(0, OFFN // 16, ccopy, 0)

    def place(g, c):
        dv = dloc_v[pl.ds(g * 16, 16)]
        iv = ids_v[pl.ds(g * 16, 16)]
        rank, lastm = plsc.scan_count(dv)
        base = plsc.load_gather(cur_v, [dv])
        pos = jnp.minimum(base + rank - SCAN_BASE, CAP - 1)
        plsc.store_scatter(sids_v, [pos], iv)
        plsc.store_scatter(cur_v, [dv], pos + 1, mask=lastm)
        return c

    lax.fori_loop(0, CAP // 16, place, 0)

    pltpu.sync_copy(sids_v, sids_hbm.at[wid])
    pltpu.sync_copy(offs_v, offs_hbm.at[wid])


_compact = pl.kernel(
    _compact_body,
    out_type=(
        jax.ShapeDtypeStruct((NW, CAP), jnp.int32),
        jax.ShapeDtypeStruct((NW, OFFN), jnp.int32),
    ),
    mesh=_sc_mesh,
    compiler_params=_sc_params,
    scratch_types=[
        pltpu.VMEM((DB,), jnp.int32),
        pltpu.VMEM((CAP,), jnp.int32),
        pltpu.VMEM((CAP,), jnp.int32),
        pltpu.VMEM((CAP,), jnp.int32),
        pltpu.VMEM((OFFN,), jnp.int32),
        pltpu.VMEM((OFFN,), jnp.int32),
        pltpu.VMEM((OFFN,), jnp.int32),
    ],
)


ZCH = 200
SUB = 40
NSUB = ZCH // SUB
NZ = EPW // ZCH


def _gather_body(dst_h, src_h, a_hbm, b_hbm, z_hbm, idxd, idxs, bufa, bufb,
                 sem, semw):
    wid = _wid()
    base = wid * EPW
    pltpu.sync_copy(dst_h.at[pl.ds(base, EPW)], idxd)
    pltpu.sync_copy(src_h.at[pl.ds(base, EPW)], idxs)

    def fire(p, slot):
        off = p * ZCH
        for q in range(NSUB):
            isl = pl.ds(off + q * SUB, SUB)
            bsl = pl.ds(q * SUB, SUB)
            pltpu.async_copy(a_hbm.at[idxd.at[isl]], bufa.at[slot, bsl],
                             sem.at[slot])
            pltpu.async_copy(b_hbm.at[idxs.at[isl]], bufb.at[slot, bsl],
                             sem.at[slot])

    def drain(p, slot):
        off = p * ZCH
        for q in range(NSUB):
            isl = pl.ds(off + q * SUB, SUB)
            bsl = pl.ds(q * SUB, SUB)
            pltpu.make_async_copy(a_hbm.at[idxd.at[isl]], bufa.at[slot, bsl],
                                  sem.at[slot]).wait()
            pltpu.make_async_copy(b_hbm.at[idxs.at[isl]], bufb.at[slot, bsl],
                                  sem.at[slot]).wait()

    def zdesc(p, slot):
        return pltpu.make_async_copy(
            bufa.at[slot], z_hbm.at[pl.ds(base + p * ZCH, ZCH)], semw.at[slot])

    fire(0, 0)

    def pair(P, c):
        for b2 in (0, 1):
            p = 2 * P + b2

            @pl.when(p + 1 < NZ)
            def _():
                @pl.when(p >= 1)
                def _():
                    zdesc(p - 1, 1 - b2).wait()

                fire(p + 1, 1 - b2)

            drain(p, b2)

            def addrow(g, c2):
                for rr in range(4):
                    r = g * 4 + rr
                    for cc in range(4):
                        sl = pl.ds(cc * 16, 16)
                        bufa[b2, r, sl] = bufa[b2, r, sl] + bufb[b2, r, sl]
                return c2

            lax.fori_loop(0, ZCH // 4, addrow, 0)
            zdesc(p, b2).start()
        return c

    lax.fori_loop(0, NZ // 2, pair, 0)
    zdesc(NZ - 2, 0).wait()
    zdesc(NZ - 1, 1).wait()


_gather = pl.kernel(
    _gather_body,
    out_type=jax.ShapeDtypeStruct((E, H), jnp.float32),
    mesh=_sc_mesh,
    compiler_params=_sc_params,
    scratch_types=[
        pltpu.VMEM((EPW,), jnp.int32),
        pltpu.VMEM((EPW,), jnp.int32),
        pltpu.VMEM((2, ZCH, H), jnp.float32),
        pltpu.VMEM((2, ZCH, H), jnp.float32),
        pltpu.SemaphoreType.DMA((2,)),
        pltpu.SemaphoreType.DMA((2,)),
    ],
)


SCH2 = 400
SSUB = 80
NSSUB = SCH2 // SSUB
NSC = CAP // SCH2


def _make_scatter(layer2: bool):
    def body(sids_hbm, offs_hbm, y_hbm, out_hbm, idxb, offs_v, orep, ybuf, acc,
             sem):
        wid = _wid()
        lo = wid * NPT_REAL
        initv = jnp.full((16,), -jnp.inf if layer2 else 0.0, jnp.float32)

        def ini(i, c):
            for cc in range(4):
                acc[i, pl.ds(cc * 16, 16)] = initv
            return c

        lax.fori_loop(0, NPT, ini, 0)

        pltpu.sync_copy(sids_hbm.at[wid], idxb)
        pltpu.sync_copy(offs_hbm.at[wid], offs_v)

        def rep(g, c):
            v = offs_v[pl.ds(g * 16, 16)]
            for u in range(16):
                orep[pl.ds(g * 256 + u * 16, 16)] = jnp.broadcast_to(v[u], (16,))
            return c

        lax.fori_loop(0, OFFN // 16, rep, 0)

        total = orep[pl.ds(DUMMY_SLOT * 16, 16)][0]

        def fire(p, slot):
            off = p * SCH2
            for q in range(NSSUB):
                isl = pl.ds(off + q * SSUB, SSUB)
                bsl = pl.ds(q * SSUB, SSUB)
                pltpu.async_copy(y_hbm.at[idxb.at[isl]], ybuf.at[slot, bsl],
                                 sem.at[slot])

        def drain(p, slot):
            off = p * SCH2
            for q in range(NSSUB):
                isl = pl.ds(off + q * SSUB, SSUB)
                bsl = pl.ds(q * SSUB, SSUB)
                pltpu.make_async_copy(y_hbm.at[idxb.at[isl]],
                                      ybuf.at[slot, bsl], sem.at[slot]).wait()

        fire(0, 0)

        def pair(P, nstate):
            for b2 in (0, 1):
                p = 2 * P + b2

                @pl.when(p + 1 < NSC)
                def _():
                    fire(p + 1, 1 - b2)

                drain(p, b2)

                cs = p * SCH2
                ce = jnp.minimum((p + 1) * SCH2, total)

                def wcond(st):
                    return st[1] < ce

                def wbody(st):
                    n, e = st
                    n_end = orep[pl.ds((n + 1) * 16, 16)][0]
                    seg_end = jnp.minimum(n_end, ce)
                    r0 = acc[n, pl.ds(0, 16)]
                    r1 = acc[n, pl.ds(16, 16)]
                    r2 = acc[n, pl.ds(32, 16)]
                    r3 = acc[n, pl.ds(48, 16)]

                    def ed(i, regs):
                        a0, a1, a2, a3 = regs
                        r = i - cs
                        a0 = jnp.maximum(a0, ybuf[b2, r, pl.ds(0, 16)])
                        a1 = jnp.maximum(a1, ybuf[b2, r, pl.ds(16, 16)])
                        a2 = jnp.maximum(a2, ybuf[b2, r, pl.ds(32, 16)])
                        a3 = jnp.maximum(a3, ybuf[b2, r, pl.ds(48, 16)])
                        return (a0, a1, a2, a3)

                    r0, r1, r2, r3 = lax.fori_loop(e, seg_end, ed,
                                                   (r0, r1, r2, r3))
                    acc[n, pl.ds(0, 16)] = r0
                    acc[n, pl.ds(16, 16)] = r1
                    acc[n, pl.ds(32, 16)] = r2
                    acc[n, pl.ds(48, 16)] = r3
                    n2 = jnp.where(seg_end == n_end,
                                   jnp.minimum(n + 1, OFFN - 2), n)
                    return (n2, seg_end)

                nstate_in = (nstate[0], jnp.maximum(nstate[1], cs))
                nstate = lax.while_loop(wcond, wbody, nstate_in)
            return nstate

        n0 = jnp.int32(0)
        e0 = jnp.int32(0)
        lax.fori_loop(0, NSC // 2, pair, (n0, e0))

        if layer2:
            ninf = jnp.float32(-jnp.inf)

            def fix(i, c):
                for cc in range(4):
                    sl = pl.ds(cc * 16, 16)
                    v = acc[i, sl]
                    acc[i, sl] = jnp.where(v == ninf, jnp.float32(0.0), v)
                return c

            lax.fori_loop(0, NPT_REAL, fix, 0)

        last = N - (NW - 1) * NPT_REAL

        @pl.when(wid == NW - 1)
        def _():
            pltpu.sync_copy(acc.at[pl.ds(0, last)], out_hbm.at[pl.ds(lo, last)])

        @pl.when(wid != NW - 1)
        def _():
            pltpu.sync_copy(acc.at[pl.ds(0, NPT_REAL)],
                            out_hbm.at[pl.ds(lo, NPT_REAL)])

    return pl.kernel(
        body,
        out_type=jax.ShapeDtypeStruct((N, H), jnp.float32),
        mesh=_sc_mesh,
        compiler_params=_sc_params,
        scratch_types=[
            pltpu.VMEM((CAP,), jnp.int32),
            pltpu.VMEM((OFFN,), jnp.int32),
            pltpu.VMEM((OFFN * 16,), jnp.int32),
            pltpu.VMEM((2, SCH2, H), jnp.float32),
            pltpu.VMEM((NPT, H), jnp.float32),
            pltpu.SemaphoreType.DMA((2,)),
        ],
    )


_scatter1 = _make_scatter(layer2=False)
_scatter2 = _make_scatter(layer2=True)


def _precompute(x, W, Din):
    def body(x_ref, w_ref, a_ref, b_ref):
        xv = x_ref[...]
        wa = w_ref[0:Din, :]
        wb = w_ref[Din:2 * Din, :]
        a_ref[...] = jnp.dot(xv, wa - wb, preferred_element_type=jnp.float32,
                            precision=lax.Precision.HIGHEST)
        b_ref[...] = jnp.dot(xv, wb, preferred_element_type=jnp.float32,
                            precision=lax.Precision.HIGHEST)

    return pl.pallas_call(
        body,
        out_shape=(
            jax.ShapeDtypeStruct((N, H), jnp.float32),
            jax.ShapeDtypeStruct((N, H), jnp.float32),
        ),
    )(x, W)


MB = 2000


def _mlp(z, b1, W2, b2):
    def body(z_ref, b1_ref, w2_ref, b2_ref, y_ref):
        h = jnp.maximum(z_ref[...] + b1_ref[...], 0.0)
        y_ref[...] = (jnp.dot(h, w2_ref[...], preferred_element_type=jnp.float32,
                             precision=lax.Precision.HIGHEST)
                      + b2_ref[...])

    return pl.pallas_call(
        body,
        grid=(E // MB,),
        in_specs=[
            pl.BlockSpec((MB, H), lambda i: (i, 0)),
            pl.BlockSpec((1, H), lambda i: (0, 0)),
            pl.BlockSpec((H, H), lambda i: (0, 0)),
            pl.BlockSpec((1, H), lambda i: (0, 0)),
        ],
        out_specs=pl.BlockSpec((MB, H), lambda i: (i, 0)),
        out_shape=jax.ShapeDtypeStruct((E, H), jnp.float32),
    )(z, b1, W2, b2)


def kernel(x, edge_index, W11, b11, W12, b12, W21, b21, W22, b22):
    src = edge_index[0]
    dst = edge_index[1]
    sids, offs = _compact(dst)

    A1, B1 = _precompute(x, W11, D)
    Z1 = _gather(dst, src, A1, B1)
    Y1 = _mlp(Z1, b11.reshape(1, H), W12, b12.reshape(1, H))
    h = _scatter1(sids, offs, Y1)

    A2, B2 = _precompute(h, W21, H)
    Z2 = _gather(dst, src, A2, B2)
    Y2 = _mlp(Z2, b21.reshape(1, H), W22, b22.reshape(1, H))
    out = _scatter2(sids, offs, Y2)
    return out
